# baseline (speedup 1.0000x reference)
import functools

import jax
import jax.numpy as jnp
from jax.experimental import pallas as pl
from jax.experimental.pallas import tpu as pltpu


def _pool_kernel(x_ref, out_ref, *, t_total, c):
    k = pl.program_id(1)
    x = x_ref[...].astype(jnp.float32)
    s = jnp.sum(x, axis=-1)
    ss = jnp.sum(x * x, axis=-1)

    @pl.when(k == 0)
    def _start():
        out_ref[:, :c] = s
        out_ref[:, c:] = ss

    @pl.when(k > 0)
    def _accumulate():
        out_ref[:, :c] += s
        out_ref[:, c:] += ss

    @pl.when(k == pl.num_programs(1) - 1)
    def _finalize():
        s_tot = out_ref[:, :c]
        mean = s_tot * (1.0 / jnp.float32(t_total))
        var = (out_ref[:, c:] - s_tot * mean) * (1.0 / jnp.float32(t_total - 1))
        std = jnp.sqrt(jnp.maximum(var, 0.0))
        out_ref[:, :c] = mean
        out_ref[:, c:] = std


def kernel(x):
    B, C, T = x.shape
    tb = 8
    tt = 256

    out = pl.pallas_call(
        functools.partial(_pool_kernel, t_total=T, c=C),
        out_shape=jax.ShapeDtypeStruct((B, 2 * C), jnp.float32),
        grid=(B // tb, T // tt),
        in_specs=[pl.BlockSpec((tb, C, tt), lambda b, k: (b, 0, k))],
        out_specs=pl.BlockSpec((tb, 2 * C), lambda b, k: (b, 0)),
        compiler_params=pltpu.CompilerParams(
            dimension_semantics=("parallel", "arbitrary"),
        ),
    )(x)
    return out.astype(x.dtype)


# speedup vs baseline: 1.2206x; 1.2206x over previous
import functools

import jax
import jax.numpy as jnp
from jax.experimental import pallas as pl
from jax.experimental.pallas import tpu as pltpu


def _pool_kernel(x_ref, out_ref, *, t_total, c):
    k = pl.program_id(1)
    x = x_ref[...].astype(jnp.float32)
    s = jnp.sum(x, axis=-1)
    ss = jnp.sum(x * x, axis=-1)

    @pl.when(k == 0)
    def _start():
        out_ref[:, :c] = s
        out_ref[:, c:] = ss

    @pl.when(k > 0)
    def _accumulate():
        out_ref[:, :c] += s
        out_ref[:, c:] += ss

    @pl.when(k == pl.num_programs(1) - 1)
    def _finalize():
        s_tot = out_ref[:, :c]
        mean = s_tot * (1.0 / jnp.float32(t_total))
        var = (out_ref[:, c:] - s_tot * mean) * (1.0 / jnp.float32(t_total - 1))
        std = jnp.sqrt(jnp.maximum(var, 0.0))
        out_ref[:, :c] = mean
        out_ref[:, c:] = std


def kernel(x):
    B, C, T = x.shape
    tb = 8
    tt = 512

    out = pl.pallas_call(
        functools.partial(_pool_kernel, t_total=T, c=C),
        out_shape=jax.ShapeDtypeStruct((B, 2 * C), jnp.float32),
        grid=(B // tb, T // tt),
        in_specs=[pl.BlockSpec((tb, C, tt), lambda b, k: (b, 0, k))],
        out_specs=pl.BlockSpec((tb, 2 * C), lambda b, k: (b, 0)),
        compiler_params=pltpu.CompilerParams(
            dimension_semantics=("parallel", "arbitrary"),
            vmem_limit_bytes=64 * 1024 * 1024,
        ),
    )(x)
    return out.astype(x.dtype)


# speedup vs baseline: 1.2207x; 1.0001x over previous
import functools

import jax
import jax.numpy as jnp
from jax.experimental import pallas as pl
from jax.experimental.pallas import tpu as pltpu


def _pool_kernel(x0_ref, x1_ref, out_ref, *, t_total, c):
    k = pl.program_id(1)
    ch = c // 2

    for h, x_ref in enumerate((x0_ref, x1_ref)):
        x = x_ref[...].astype(jnp.float32)
        s = jnp.sum(x, axis=-1)
        ss = jnp.sum(x * x, axis=-1)
        mcols = pl.ds(h * ch, ch)
        scols = pl.ds(c + h * ch, ch)

        @pl.when(k == 0)
        def _start():
            out_ref[:, mcols] = s
            out_ref[:, scols] = ss

        @pl.when(k > 0)
        def _accumulate():
            out_ref[:, mcols] += s
            out_ref[:, scols] += ss

    @pl.when(k == pl.num_programs(1) - 1)
    def _finalize():
        s_tot = out_ref[:, :c]
        mean = s_tot * (1.0 / jnp.float32(t_total))
        var = (out_ref[:, c:] - s_tot * mean) * (1.0 / jnp.float32(t_total - 1))
        std = jnp.sqrt(jnp.maximum(var, 0.0))
        out_ref[:, :c] = mean
        out_ref[:, c:] = std


def kernel(x):
    B, C, T = x.shape
    tb = 8
    tt = 512

    out = pl.pallas_call(
        functools.partial(_pool_kernel, t_total=T, c=C),
        out_shape=jax.ShapeDtypeStruct((B, 2 * C), jnp.float32),
        grid=(B // tb, T // tt),
        in_specs=[
            pl.BlockSpec((tb, C // 2, tt), lambda b, k: (b, 0, k)),
            pl.BlockSpec((tb, C // 2, tt), lambda b, k: (b, 1, k)),
        ],
        out_specs=pl.BlockSpec((tb, 2 * C), lambda b, k: (b, 0)),
        compiler_params=pltpu.CompilerParams(
            dimension_semantics=("parallel", "arbitrary"),
            vmem_limit_bytes=64 * 1024 * 1024,
        ),
    )(x, x)
    return out.astype(x.dtype)


# speedup vs baseline: 1.2211x; 1.0004x over previous
import functools

import jax
import jax.numpy as jnp
from jax.experimental import pallas as pl
from jax.experimental.pallas import tpu as pltpu


def _pool_kernel(x_ref, out_ref, *, t_total, c):
    k = pl.program_id(1)
    x = x_ref[...].astype(jnp.float32)
    s = jnp.sum(x, axis=-1)
    ss = jnp.sum(x * x, axis=-1)

    @pl.when(k == 0)
    def _start():
        out_ref[:, :c] = s
        out_ref[:, c:] = ss

    @pl.when(k > 0)
    def _accumulate():
        out_ref[:, :c] += s
        out_ref[:, c:] += ss

    @pl.when(k == pl.num_programs(1) - 1)
    def _finalize():
        s_tot = out_ref[:, :c]
        mean = s_tot * (1.0 / jnp.float32(t_total))
        var = (out_ref[:, c:] - s_tot * mean) * (1.0 / jnp.float32(t_total - 1))
        std = jnp.sqrt(jnp.maximum(var, 0.0))
        out_ref[:, :c] = mean
        out_ref[:, c:] = std


def kernel(x):
    B, C, T = x.shape
    tb = 8
    tt = 512

    out = pl.pallas_call(
        functools.partial(_pool_kernel, t_total=T, c=C),
        out_shape=jax.ShapeDtypeStruct((B, 2 * C), jnp.float32),
        grid=(B // tb, T // tt),
        in_specs=[pl.BlockSpec((tb, C, tt), lambda b, k: (b, 0, k))],
        out_specs=pl.BlockSpec((tb, 2 * C), lambda b, k: (b, 0)),
        compiler_params=pltpu.CompilerParams(
            dimension_semantics=("parallel", "arbitrary"),
            vmem_limit_bytes=64 * 1024 * 1024,
        ),
    )(x)
    return out.astype(x.dtype)
